# baseline (speedup 1.0000x reference)
import jax
import jax.numpy as jnp
from jax import lax
from jax.experimental import pallas as pl
from jax.experimental.pallas import tpu as pltpu
from jax.experimental.pallas import tpu_sc as plsc

VOCAB = 30522
HID = 768
PROMPT = 20
B = 32
S = 512
EPS = 1e-12
NW = 32
SW = S // NW
NBUF = 4
NC = 4
HB = B // NC
PBLK = 32


def _sc_gather_body(ids_hbm, word_hbm, out_hbm, idx_v, b0, b1, b2, b3,
                    g0, g1, g2, g3, w0, w1, w2, w3):
    buf = (b0, b1, b2, b3)
    gsem = (g0, g1, g2, g3)
    wsem = (w0, w1, w2, w3)

    cid = lax.axis_index("c")
    sid = lax.axis_index("s")
    wid = sid * 2 + cid
    s0 = wid * SW

    pltpu.sync_copy(ids_hbm.at[pl.ds(wid * (HB * SW), HB * SW)], idx_v)

    def _gather(b, k):
        return pltpu.make_async_copy(
            word_hbm.at[idx_v.at[pl.ds(b * SW, SW)]], buf[k], gsem[k])

    def _write(b, k):
        return pltpu.make_async_copy(
            buf[k], out_hbm.at[b, pl.ds(s0, SW)], wsem[k])

    _gather(0, 0).start()
    _gather(1, 1).start()

    def _quad(g, c):
        for k in range(NBUF):
            b = g * NBUF + k
            _gather(b, k).wait()
            _write(b, k).start()

            kk = (k + 2) % NBUF

            @pl.when(b + 2 < HB)
            def _():
                @pl.when(b >= 2)
                def _():
                    _write(b - 2, kk).wait()
                _gather(b + 2, kk).start()
        return c
    lax.fori_loop(0, HB // NBUF, _quad, 0)

    for b in range(HB - NBUF, HB):
        _write(b, b % NBUF).wait()


def _make_tc_body(aliased):
    def _tc_ln_body(*refs):
        if aliased:
            _, inter_ref, pos_ref, pshift_ref, gamma_ref, beta_ref, out_ref = refs
        else:
            inter_ref, pos_ref, pshift_ref, gamma_ref, beta_ref, out_ref = refs
        x = inter_ref[0]
        row = lax.broadcasted_iota(jnp.int32, (PBLK, 1), 0)
        pmask = jnp.logical_and(row >= 1, row < 1 + PROMPT)
        head = jnp.where(pmask, pshift_ref[...], x[0:PBLK])
        x = jnp.concatenate([head, x[PBLK:]], axis=0)
        x = x + pos_ref[...]
        mean = jnp.mean(x, axis=-1, keepdims=True)
        xc = x - mean
        var = jnp.mean(xc * xc, axis=-1, keepdims=True)
        y = xc * lax.rsqrt(var + EPS)
        out_ref[0] = y * gamma_ref[...][None, :] + beta_ref[...][None, :]
    return _tc_ln_body


def kernel(input_ids, word_emb, pos_emb, type_emb, prompt_emb, gamma, beta):
    ids_r = input_ids.reshape(B, NW, SW).transpose(1, 0, 2)

    mesh = plsc.VectorSubcoreMesh(core_axis_name="c", subcore_axis_name="s")
    sc_gather = pl.kernel(
        _sc_gather_body,
        out_type=jax.ShapeDtypeStruct((HB, S, HID), jnp.float32),
        mesh=mesh,
        scratch_types=(
            [pltpu.VMEM((HB * SW,), jnp.int32)]
            + [pltpu.VMEM((SW, HID), jnp.float32)] * NBUF
            + [pltpu.SemaphoreType.DMA] * (2 * NBUF)
        ),
    )
    inters = [
        sc_gather(ids_r[:, i * HB:(i + 1) * HB].reshape(NW * HB * SW),
                  word_emb)
        for i in range(NC)
    ]

    pos2 = pos_emb + type_emb[0][None, :]
    pshift = jnp.zeros((PBLK, HID), jnp.float32).at[1:1 + PROMPT].set(
        prompt_emb)

    data_specs = [
        pl.BlockSpec((S, HID), lambda b: (0, 0)),
        pl.BlockSpec((PBLK, HID), lambda b: (0, 0)),
        pl.BlockSpec((HID,), lambda b: (0,)),
        pl.BlockSpec((HID,), lambda b: (0,)),
    ]

    out = None
    for i in range(NC):
        def _out_idx(b, _i=i):
            return (b + _i * HB, 0, 0)
        if i == 0:
            out = pl.pallas_call(
                _make_tc_body(False),
                out_shape=jax.ShapeDtypeStruct((B, S, HID), jnp.float32),
                grid=(HB,),
                in_specs=([pl.BlockSpec((1, S, HID), lambda b: (b, 0, 0))]
                          + data_specs),
                out_specs=pl.BlockSpec((1, S, HID), _out_idx),
            )(inters[0], pos2, pshift, gamma, beta)
        else:
            out = pl.pallas_call(
                _make_tc_body(True),
                out_shape=jax.ShapeDtypeStruct((B, S, HID), jnp.float32),
                grid=(HB,),
                in_specs=([pl.BlockSpec(memory_space=pl.ANY)]
                          + [pl.BlockSpec((1, S, HID), lambda b: (b, 0, 0))]
                          + data_specs),
                out_specs=pl.BlockSpec((1, S, HID), _out_idx),
                input_output_aliases={0: 0},
            )(out, inters[i], pos2, pshift, gamma, beta)
    return out


# speedup vs baseline: 1.0150x; 1.0150x over previous
import jax
import jax.numpy as jnp
from jax import lax
from jax.experimental import pallas as pl
from jax.experimental.pallas import tpu as pltpu
from jax.experimental.pallas import tpu_sc as plsc

VOCAB = 30522
HID = 768
PROMPT = 20
B = 32
S = 512
EPS = 1e-12
NW = 32
SW = S // NW
NBUF = 4
CHUNKS = (8, 24)
PBLK = 32


def _make_sc_body(cb):
    def _sc_gather_body(ids_hbm, word_hbm, out_hbm, idx_v, b0, b1, b2, b3,
                        g0, g1, g2, g3, w0, w1, w2, w3):
        buf = (b0, b1, b2, b3)
        gsem = (g0, g1, g2, g3)
        wsem = (w0, w1, w2, w3)

        cid = lax.axis_index("c")
        sid = lax.axis_index("s")
        wid = sid * 2 + cid
        s0 = wid * SW

        pltpu.sync_copy(ids_hbm.at[pl.ds(wid * (cb * SW), cb * SW)], idx_v)

        def _gather(b, k):
            return pltpu.make_async_copy(
                word_hbm.at[idx_v.at[pl.ds(b * SW, SW)]], buf[k], gsem[k])

        def _write(b, k):
            return pltpu.make_async_copy(
                buf[k], out_hbm.at[b, pl.ds(s0, SW)], wsem[k])

        _gather(0, 0).start()
        _gather(1, 1).start()

        def _quad(g, c):
            for k in range(NBUF):
                b = g * NBUF + k
                _gather(b, k).wait()
                _write(b, k).start()

                kk = (k + 2) % NBUF

                @pl.when(b + 2 < cb)
                def _():
                    @pl.when(b >= 2)
                    def _():
                        _write(b - 2, kk).wait()
                    _gather(b + 2, kk).start()
            return c
        lax.fori_loop(0, cb // NBUF, _quad, 0)

        for b in range(cb - NBUF, cb):
            _write(b, b % NBUF).wait()
    return _sc_gather_body


def _make_tc_body(aliased):
    def _tc_ln_body(*refs):
        if aliased:
            _, inter_ref, pos_ref, pshift_ref, gamma_ref, beta_ref, out_ref = refs
        else:
            inter_ref, pos_ref, pshift_ref, gamma_ref, beta_ref, out_ref = refs
        x = inter_ref[0]
        row = lax.broadcasted_iota(jnp.int32, (PBLK, 1), 0)
        pmask = jnp.logical_and(row >= 1, row < 1 + PROMPT)
        head = jnp.where(pmask, pshift_ref[...], x[0:PBLK])
        x = jnp.concatenate([head, x[PBLK:]], axis=0)
        x = x + pos_ref[...]
        mean = jnp.mean(x, axis=-1, keepdims=True)
        xc = x - mean
        var = jnp.mean(xc * xc, axis=-1, keepdims=True)
        y = xc * lax.rsqrt(var + EPS)
        out_ref[0] = y * gamma_ref[...][None, :] + beta_ref[...][None, :]
    return _tc_ln_body


def kernel(input_ids, word_emb, pos_emb, type_emb, prompt_emb, gamma, beta):
    ids_r = input_ids.reshape(B, NW, SW).transpose(1, 0, 2)

    mesh = plsc.VectorSubcoreMesh(core_axis_name="c", subcore_axis_name="s")
    inters = []
    off = 0
    for cb in CHUNKS:
        sc_gather = pl.kernel(
            _make_sc_body(cb),
            out_type=jax.ShapeDtypeStruct((cb, S, HID), jnp.float32),
            mesh=mesh,
            scratch_types=(
                [pltpu.VMEM((cb * SW,), jnp.int32)]
                + [pltpu.VMEM((SW, HID), jnp.float32)] * NBUF
                + [pltpu.SemaphoreType.DMA] * (2 * NBUF)
            ),
        )
        inters.append(sc_gather(
            ids_r[:, off:off + cb].reshape(NW * cb * SW), word_emb))
        off += cb

    pos2 = pos_emb + type_emb[0][None, :]
    pshift = jnp.zeros((PBLK, HID), jnp.float32).at[1:1 + PROMPT].set(
        prompt_emb)

    data_specs = [
        pl.BlockSpec((S, HID), lambda b: (0, 0)),
        pl.BlockSpec((PBLK, HID), lambda b: (0, 0)),
        pl.BlockSpec((HID,), lambda b: (0,)),
        pl.BlockSpec((HID,), lambda b: (0,)),
    ]

    out = None
    off = 0
    for i, cb in enumerate(CHUNKS):
        def _out_idx(b, _o=off):
            return (b + _o, 0, 0)
        if i == 0:
            out = pl.pallas_call(
                _make_tc_body(False),
                out_shape=jax.ShapeDtypeStruct((B, S, HID), jnp.float32),
                grid=(cb,),
                in_specs=([pl.BlockSpec((1, S, HID), lambda b: (b, 0, 0))]
                          + data_specs),
                out_specs=pl.BlockSpec((1, S, HID), _out_idx),
            )(inters[0], pos2, pshift, gamma, beta)
        else:
            out = pl.pallas_call(
                _make_tc_body(True),
                out_shape=jax.ShapeDtypeStruct((B, S, HID), jnp.float32),
                grid=(cb,),
                in_specs=([pl.BlockSpec(memory_space=pl.ANY)]
                          + [pl.BlockSpec((1, S, HID), lambda b: (b, 0, 0))]
                          + data_specs),
                out_specs=pl.BlockSpec((1, S, HID), _out_idx),
                input_output_aliases={0: 0},
            )(out, inters[i], pos2, pshift, gamma, beta)
        off += cb
    return out


# speedup vs baseline: 1.0387x; 1.0233x over previous
import jax
import jax.numpy as jnp
from jax import lax
from jax.experimental import pallas as pl
from jax.experimental.pallas import tpu as pltpu
from jax.experimental.pallas import tpu_sc as plsc

VOCAB = 30522
HID = 768
PROMPT = 20
B = 32
S = 512
EPS = 1e-12
NW = 32
SW = S // NW
NBUF = 4
CHUNKS = (16, 16)
PBLK = 32


def _make_sc_body(cb):
    def _sc_gather_body(ids_hbm, word_hbm, out_hbm, idx_v, b0, b1, b2, b3,
                        g0, g1, g2, g3, w0, w1, w2, w3):
        buf = (b0, b1, b2, b3)
        gsem = (g0, g1, g2, g3)
        wsem = (w0, w1, w2, w3)

        cid = lax.axis_index("c")
        sid = lax.axis_index("s")
        wid = sid * 2 + cid
        s0 = wid * SW

        pltpu.sync_copy(ids_hbm.at[pl.ds(wid * (cb * SW), cb * SW)], idx_v)

        def _gather(b, k):
            return pltpu.make_async_copy(
                word_hbm.at[idx_v.at[pl.ds(b * SW, SW)]], buf[k], gsem[k])

        def _write(b, k):
            return pltpu.make_async_copy(
                buf[k], out_hbm.at[b, pl.ds(s0, SW)], wsem[k])

        _gather(0, 0).start()
        _gather(1, 1).start()

        def _quad(g, c):
            for k in range(NBUF):
                b = g * NBUF + k
                _gather(b, k).wait()
                _write(b, k).start()

                kk = (k + 2) % NBUF

                @pl.when(b + 2 < cb)
                def _():
                    @pl.when(b >= 2)
                    def _():
                        _write(b - 2, kk).wait()
                    _gather(b + 2, kk).start()
            return c
        lax.fori_loop(0, cb // NBUF, _quad, 0)

        for b in range(cb - NBUF, cb):
            _write(b, b % NBUF).wait()
    return _sc_gather_body


def _make_tc_body(aliased):
    def _tc_ln_body(*refs):
        if aliased:
            _, inter_ref, pos_ref, pshift_ref, gamma_ref, beta_ref, out_ref = refs
        else:
            inter_ref, pos_ref, pshift_ref, gamma_ref, beta_ref, out_ref = refs
        x = inter_ref[0]
        row = lax.broadcasted_iota(jnp.int32, (PBLK, 1), 0)
        pmask = jnp.logical_and(row >= 1, row < 1 + PROMPT)
        head = jnp.where(pmask, pshift_ref[...], x[0:PBLK])
        x = jnp.concatenate([head, x[PBLK:]], axis=0)
        x = x + pos_ref[...]
        mean = jnp.mean(x, axis=-1, keepdims=True)
        xc = x - mean
        var = jnp.mean(xc * xc, axis=-1, keepdims=True)
        y = xc * lax.rsqrt(var + EPS)
        out_ref[0] = y * gamma_ref[...][None, :] + beta_ref[...][None, :]
    return _tc_ln_body


def kernel(input_ids, word_emb, pos_emb, type_emb, prompt_emb, gamma, beta):
    ids_r = input_ids.reshape(B, NW, SW).transpose(1, 0, 2)

    mesh = plsc.VectorSubcoreMesh(core_axis_name="c", subcore_axis_name="s")
    inters = []
    off = 0
    for cb in CHUNKS:
        sc_gather = pl.kernel(
            _make_sc_body(cb),
            out_type=jax.ShapeDtypeStruct((cb, S, HID), jnp.float32),
            mesh=mesh,
            scratch_types=(
                [pltpu.VMEM((cb * SW,), jnp.int32)]
                + [pltpu.VMEM((SW, HID), jnp.float32)] * NBUF
                + [pltpu.SemaphoreType.DMA] * (2 * NBUF)
            ),
        )
        inters.append(sc_gather(
            ids_r[:, off:off + cb].reshape(NW * cb * SW), word_emb))
        off += cb

    pos2 = pos_emb + type_emb[0][None, :]
    pshift = jnp.zeros((PBLK, HID), jnp.float32).at[1:1 + PROMPT].set(
        prompt_emb)

    data_specs = [
        pl.BlockSpec((S, HID), lambda b: (0, 0)),
        pl.BlockSpec((PBLK, HID), lambda b: (0, 0)),
        pl.BlockSpec((HID,), lambda b: (0,)),
        pl.BlockSpec((HID,), lambda b: (0,)),
    ]

    out = None
    off = 0
    for i, cb in enumerate(CHUNKS):
        def _out_idx(b, _o=off):
            return (b + _o, 0, 0)
        if i == 0:
            out = pl.pallas_call(
                _make_tc_body(False),
                out_shape=jax.ShapeDtypeStruct((B, S, HID), jnp.float32),
                grid=(cb,),
                in_specs=([pl.BlockSpec((1, S, HID), lambda b: (b, 0, 0))]
                          + data_specs),
                out_specs=pl.BlockSpec((1, S, HID), _out_idx),
            )(inters[0], pos2, pshift, gamma, beta)
        else:
            out = pl.pallas_call(
                _make_tc_body(True),
                out_shape=jax.ShapeDtypeStruct((B, S, HID), jnp.float32),
                grid=(cb,),
                in_specs=([pl.BlockSpec(memory_space=pl.ANY)]
                          + [pl.BlockSpec((1, S, HID), lambda b: (b, 0, 0))]
                          + data_specs),
                out_specs=pl.BlockSpec((1, S, HID), _out_idx),
                input_output_aliases={0: 0},
            )(out, inters[i], pos2, pshift, gamma, beta)
        off += cb
    return out
